# baseline (speedup 1.0000x reference)
import jax
import jax.numpy as jnp
from jax.experimental import pallas as pl
from jax.experimental.pallas import tpu as pltpu
from jax.experimental.pallas import tpu_sc as plsc

_W = 128
_EP = 256


def kernel(table, inputs):
    B, L = inputs.shape
    V, E = table.shape
    n = B * L
    m = 2 * n

    tablep = jnp.pad(table, ((0, 0), (0, _EP - E))).reshape(2 * V, 128)
    iv = inputs.reshape(-1)
    idx2 = (2 * iv[:, None] + jnp.arange(2, dtype=iv.dtype)).reshape(1, m)

    mesh = plsc.VectorSubcoreMesh(core_axis_name="core",
                                  subcore_axis_name="subcore")

    @pl.kernel(out_type=jax.ShapeDtypeStruct((m, 128), table.dtype), mesh=mesh)
    def gather_kernel(table_hbm, idx_hbm, out_hbm):
        def body(idx_vmem, out_vmem):
            pltpu.sync_copy(table_hbm.at[idx_vmem.at[0]], out_vmem)

        pltpu.emit_pipeline(
            body,
            grid=(m // _W,),
            in_specs=[pl.BlockSpec((1, _W), lambda i: (0, i))],
            out_specs=[pl.BlockSpec((_W, 128), lambda i: (i, 0))],
            core_axis_name=("core", "subcore"),
            dimension_semantics=(pltpu.PARALLEL,),
        )(idx_hbm, out_hbm)

    out = gather_kernel(tablep, idx2)
    return out.reshape(n, _EP)[:, :E].reshape(B, L * E)


# speedup vs baseline: 1.2651x; 1.2651x over previous
import jax
import jax.numpy as jnp
from jax import lax
from jax.experimental import pallas as pl
from jax.experimental.pallas import tpu as pltpu
from jax.experimental.pallas import tpu_sc as plsc

_NW = 32


def kernel(table, inputs):
    B, L = inputs.shape
    V, E = table.shape
    rows_per = B // _NW

    mesh = plsc.VectorSubcoreMesh(core_axis_name="core",
                                  subcore_axis_name="subcore")

    @pl.kernel(out_type=jax.ShapeDtypeStruct((B, L * E), table.dtype),
               mesh=mesh,
               compiler_params=pltpu.CompilerParams(use_tc_tiling_on_sc=False),
               scratch_types=[pltpu.VMEM((rows_per, L), inputs.dtype),
                              pltpu.VMEM((L, E), table.dtype),
                              pltpu.VMEM((L, E), table.dtype),
                              pltpu.SemaphoreType.DMA,
                              pltpu.SemaphoreType.DMA])
    def gather_kernel(table_hbm, idx_hbm, out_hbm, ibuf, stg0, stg1,
                      sem0, sem1):
        wid = lax.axis_index("subcore") * 2 + lax.axis_index("core")
        base = wid * rows_per
        pltpu.sync_copy(idx_hbm.at[pl.ds(base, rows_per)], ibuf)

        @pl.loop(0, rows_per // 2)
        def _(s):
            r0 = base + 2 * s
            pltpu.sync_copy(table_hbm.at[ibuf.at[2 * s]], stg0)
            h0 = [pltpu.async_copy(stg0.at[t],
                                   out_hbm.at[r0, pl.ds(t * E, E)], sem0)
                  for t in range(L)]
            pltpu.sync_copy(table_hbm.at[ibuf.at[2 * s + 1]], stg1)
            h1 = [pltpu.async_copy(stg1.at[t],
                                   out_hbm.at[r0 + 1, pl.ds(t * E, E)], sem1)
                  for t in range(L)]
            for h in h0:
                h.wait()
            for h in h1:
                h.wait()

    return gather_kernel(table, inputs)


# speedup vs baseline: 1.3724x; 1.0848x over previous
import jax
import jax.numpy as jnp
from jax.experimental import pallas as pl
from jax.experimental.pallas import tpu as pltpu
from jax.experimental.pallas import tpu_sc as plsc

_W = 128
_EP = 256


def kernel(table, inputs):
    B, L = inputs.shape
    V, E = table.shape
    n = B * L
    idx = inputs.reshape(1, n)
    tablep = jnp.pad(table, ((0, 0), (0, _EP - E)))

    tablep = pltpu.with_memory_space_constraint(tablep, pltpu.MemorySpace.HBM)
    idx = pltpu.with_memory_space_constraint(idx, pltpu.MemorySpace.HBM)

    mesh = plsc.VectorSubcoreMesh(core_axis_name="core",
                                  subcore_axis_name="subcore")

    @pl.kernel(out_type=jax.ShapeDtypeStruct((n, _EP), table.dtype), mesh=mesh)
    def gather_kernel(table_hbm, idx_hbm, out_hbm):
        def body(idx_vmem, out_vmem):
            pltpu.sync_copy(table_hbm.at[idx_vmem.at[0]], out_vmem)

        pltpu.emit_pipeline(
            body,
            grid=(n // _W,),
            in_specs=[pl.BlockSpec((1, _W), lambda i: (0, i))],
            out_specs=[pl.BlockSpec((_W, _EP), lambda i: (i, 0))],
            core_axis_name=("core", "subcore"),
            dimension_semantics=(pltpu.PARALLEL,),
        )(idx_hbm, out_hbm)

    out = gather_kernel(tablep, idx)
    return out[:, :E].reshape(B, L * E)


# speedup vs baseline: 1.3758x; 1.0025x over previous
import jax
import jax.numpy as jnp
from jax.experimental import pallas as pl
from jax.experimental.pallas import tpu as pltpu
from jax.experimental.pallas import tpu_sc as plsc

_W = 128


def kernel(table, inputs):
    B, L = inputs.shape
    V, E = table.shape
    n = B * L
    idx = inputs.reshape(1, n)

    mesh = plsc.VectorSubcoreMesh(core_axis_name="core",
                                  subcore_axis_name="subcore")

    @pl.kernel(out_type=jax.ShapeDtypeStruct((n, E), table.dtype), mesh=mesh,
               compiler_params=pltpu.CompilerParams(use_tc_tiling_on_sc=False))
    def gather_kernel(table_hbm, idx_hbm, out_hbm):
        def body(idx_vmem, out_vmem):
            pltpu.sync_copy(table_hbm.at[idx_vmem.at[0]], out_vmem)

        pltpu.emit_pipeline(
            body,
            grid=(n // _W,),
            in_specs=[pl.BlockSpec((1, _W), lambda i: (0, i))],
            out_specs=[pl.BlockSpec((_W, E), lambda i: (i, 0))],
            core_axis_name=("core", "subcore"),
            dimension_semantics=(pltpu.PARALLEL,),
        )(idx_hbm, out_hbm)

    out = gather_kernel(table, idx)
    return out.reshape(B, L * E)
